# baseline (speedup 1.0000x reference)
import functools

import jax
import jax.numpy as jnp
from jax import lax
from jax.experimental import pallas as pl
from jax.experimental.pallas import tpu as pltpu
from jax.experimental.pallas import tpu_sc as plsc

VOCAB = 1000000
D = 64
B = 4096
S = 200
NC = 2
NS = 16
NW = NC * NS
ROWS = B * S
RPW = ROWS // NW
CB = 4
CR = CB * S
NCHUNK = RPW // CR
GSZ = 100
NG = CR // GSZ


def _sc_body(idx_hbm, table_hbm, pos_hbm, out_hbm, idx_v, rows_v, pos_v, gsem):
    c = lax.axis_index("c")
    s = lax.axis_index("s")
    wid = s * NC + c
    base = wid * RPW
    pltpu.sync_copy(pos_hbm, pos_v)

    def chunk_body(g, carry):
        row0 = pl.multiple_of(base + g * CR, CR)
        pltpu.sync_copy(idx_hbm.at[pl.ds(pl.multiple_of(row0 // GSZ, NG), NG)], idx_v)
        cps = [
            pltpu.async_copy(
                table_hbm.at[idx_v.at[j]],
                rows_v.at[pl.ds(j * GSZ, GSZ)],
                gsem,
            )
            for j in range(NG)
        ]
        for cp in cps:
            cp.wait()

        def add_body(p, c2):
            pv = [pos_v[p, pl.ds(16 * k, 16)] for k in range(D // 16)]
            for i in range(CB):
                r = i * S + p
                for k in range(D // 16):
                    rows_v[r, pl.ds(16 * k, 16)] = (
                        rows_v[r, pl.ds(16 * k, 16)] + pv[k]
                    )
            return c2

        lax.fori_loop(0, S, add_body, 0)
        pltpu.sync_copy(rows_v, out_hbm.at[pl.ds(row0, CR)])
        return carry

    lax.fori_loop(0, NCHUNK, chunk_body, 0)


@functools.partial(jax.jit, static_argnames=())
def _run(idx2d, table, pos2d):
    mesh = plsc.VectorSubcoreMesh(core_axis_name="c", subcore_axis_name="s")
    f = functools.partial(
        pl.kernel,
        out_type=jax.ShapeDtypeStruct((ROWS, D), jnp.float32),
        mesh=mesh,
        scratch_types=[
            pltpu.VMEM((NG, GSZ), jnp.int32),
            pltpu.VMEM((CR, D), jnp.float32),
            pltpu.VMEM((S, D), jnp.float32),
            pltpu.SemaphoreType.DMA,
        ],
        compiler_params=pltpu.CompilerParams(use_tc_tiling_on_sc=False),
    )(_sc_body)
    return f(idx2d, table, pos2d)


def kernel(INPUT, embedding_table, positional_encoding):
    idx2d = INPUT.reshape(ROWS // GSZ, GSZ)
    pos2d = positional_encoding[0, :S, :]
    out = _run(idx2d, embedding_table, pos2d)
    return out.reshape(B, S, D)


# speedup vs baseline: 1.0690x; 1.0690x over previous
import functools

import jax
import jax.numpy as jnp
from jax import lax
from jax.experimental import pallas as pl
from jax.experimental.pallas import tpu as pltpu
from jax.experimental.pallas import tpu_sc as plsc

VOCAB = 1000000
D = 64
B = 4096
S = 200
NC = 2
NS = 16
NW = NC * NS
ROWS = B * S
RPW = ROWS // NW
CB = 2
CR = CB * S
NCHUNK = RPW // CR
GSZ = 100
NG = CR // GSZ
IPW = RPW // GSZ
NV = D // 16


def _sc_body(idx_hbm, table_hbm, pos_hbm, out_hbm,
             idx_v, rows0, rows1, pos_v, sem0, sem1):
    c = lax.axis_index("c")
    s = lax.axis_index("s")
    wid = s * NC + c
    base = wid * RPW
    pltpu.sync_copy(pos_hbm, pos_v)
    pltpu.sync_copy(idx_hbm.at[pl.ds(pl.multiple_of(wid * IPW, IPW), IPW)],
                    idx_v)

    def start(rows, sem, g):
        t0 = g * NG
        for j in range(NG):
            pltpu.async_copy(
                table_hbm.at[idx_v.at[t0 + j]],
                rows.at[pl.ds(j * GSZ, GSZ)],
                sem,
            )

    def finish(rows, sem, g):
        pltpu.make_async_copy(table_hbm.at[pl.ds(0, CR)], rows, sem).wait()

        def add_body(p):
            pv = [pos_v[p, pl.ds(16 * k, 16)] for k in range(NV)]
            for i in range(CB):
                r = i * S + p
                for k in range(NV):
                    rows[r, pl.ds(16 * k, 16)] = (
                        rows[r, pl.ds(16 * k, 16)] + pv[k]
                    )

        plsc.parallel_loop(0, S, unroll=4)(add_body)
        row0 = pl.multiple_of(base + g * CR, CR)
        pltpu.sync_copy(rows, out_hbm.at[pl.ds(row0, CR)])

    start(rows0, sem0, 0)

    def pair_body(h, carry):
        g0 = 2 * h
        start(rows1, sem1, g0 + 1)
        finish(rows0, sem0, g0)
        start(rows0, sem0, g0 + 2)
        finish(rows1, sem1, g0 + 1)
        return carry

    lax.fori_loop(0, NCHUNK // 2 - 1, pair_body, 0)
    start(rows1, sem1, NCHUNK - 1)
    finish(rows0, sem0, NCHUNK - 2)
    finish(rows1, sem1, NCHUNK - 1)


@jax.jit
def _run(idx2d, table, pos2d):
    mesh = plsc.VectorSubcoreMesh(core_axis_name="c", subcore_axis_name="s")
    f = functools.partial(
        pl.kernel,
        out_type=jax.ShapeDtypeStruct((ROWS, D), jnp.float32),
        mesh=mesh,
        scratch_types=[
            pltpu.VMEM((IPW, GSZ), jnp.int32),
            pltpu.VMEM((CR, D), jnp.float32),
            pltpu.VMEM((CR, D), jnp.float32),
            pltpu.VMEM((S, D), jnp.float32),
            pltpu.SemaphoreType.DMA,
            pltpu.SemaphoreType.DMA,
        ],
        compiler_params=pltpu.CompilerParams(use_tc_tiling_on_sc=False),
    )(_sc_body)
    return f(idx2d, table, pos2d)


def kernel(INPUT, embedding_table, positional_encoding):
    idx2d = INPUT.reshape(ROWS // GSZ, GSZ)
    pos2d = positional_encoding[0, :S, :]
    out = _run(idx2d, embedding_table, pos2d)
    return out.reshape(B, S, D)
